# baseline (speedup 1.0000x reference)
import functools
import math

import jax
import jax.numpy as jnp
import numpy as np
from jax import lax
from jax.experimental import pallas as pl
from jax.experimental.pallas import tpu as pltpu
from jax.experimental.pallas import tpu_sc as plsc

F32 = jnp.float32

NC = 2
NS = 16
LANES = 16
CH = 128


def _pad16(d: int) -> int:
    return ((d + 15) // 16) * 16


def _mesh():
    return plsc.VectorSubcoreMesh(core_axis_name="c", subcore_axis_name="s")


_SPMEM_WORDS = 2097151


def _pick_group(t, acc_fp, row_fp, npad, n_idx, n_rows, fixed_words):
    budget = (_SPMEM_WORDS - npad * acc_fp) // NS - 64
    epw = t * CH
    for kch in (2560, 1280, 640, 512, 320, 256, 128):
        if epw % kch:
            continue
        g = epw // kch
        if g < 2 or g % 2:
            continue
        need = n_rows * kch * row_fp + n_idx * epw + fixed_words
        if need <= budget:
            return kch, g
    raise ValueError("no feasible group size")


def _make_sc_degree(n, t):
    npad = n + 8
    rpt = n // NS
    kch, g = _pick_group(t, LANES, LANES, npad, 1, 1, rpt * LANES)

    @functools.partial(
        pl.kernel,
        out_type=jax.ShapeDtypeStruct((NC, n, LANES), F32),
        mesh=_mesh(),
        compiler_params=pltpu.CompilerParams(use_tc_tiling_on_sc=False),
        scratch_types=[
            pltpu.VMEM((g, kch), jnp.int32),
            pltpu.VMEM((kch, LANES), F32),
            pltpu.VMEM((rpt, LANES), F32),
            pltpu.VMEM_SHARED((npad, LANES), F32),
            pltpu.SemaphoreType.DMA,
        ],
    )
    def deg_kernel(dst_hbm, zeros_hbm, ones_hbm, out_hbm,
                   dst_v, ones_v, bounce_v, acc_sh, sem):
        c = lax.axis_index("c")
        s = lax.axis_index("s")
        wid = c * NS + s
        pltpu.sync_copy(dst_hbm.at[wid], dst_v)
        pltpu.sync_copy(ones_hbm, ones_v)
        sl = pl.ds(s * rpt, rpt)
        pltpu.sync_copy(zeros_hbm.at[sl], bounce_v)
        pltpu.sync_copy(bounce_v, acc_sh.at[sl])
        plsc.subcore_barrier()

        def fire(gg, carry):
            pltpu.make_async_copy(ones_v, acc_sh.at[dst_v.at[gg]], sem).start(
                add=True)
            return carry

        def drain(gg, carry):
            pltpu.make_async_copy(ones_v, acc_sh.at[dst_v.at[gg]], sem).wait()
            return carry

        lax.fori_loop(0, g, fire, 0)
        lax.fori_loop(0, g, drain, 0)
        plsc.subcore_barrier()
        pltpu.sync_copy(acc_sh.at[sl], bounce_v)
        pltpu.sync_copy(bounce_v, out_hbm.at[c, sl])

    return deg_kernel


def _agg_bounce_rows(t, fp, npad):
    rpt_all = [125, 25]
    for rb in rpt_all:
        try:
            kch, g = _pick_group(t, fp, fp, npad, 2, 2, rb * fp)
            return rb, kch, g
        except ValueError:
            continue
    raise ValueError("no feasible bounce/group config")


def _make_sc_agg(n, t, fp):
    npad = n + 8
    rpt = n // NS
    rb, kch, g = _agg_bounce_rows(t, fp, npad)
    nbp = rpt // rb

    @functools.partial(
        pl.kernel,
        out_type=jax.ShapeDtypeStruct((NC, n, fp), F32),
        mesh=_mesh(),
        compiler_params=pltpu.CompilerParams(use_tc_tiling_on_sc=False),
        scratch_types=[
            pltpu.VMEM((g, kch), jnp.int32),
            pltpu.VMEM((g, kch), jnp.int32),
            pltpu.VMEM((kch, fp), F32),
            pltpu.VMEM((kch, fp), F32),
            pltpu.VMEM((rb, fp), F32),
            pltpu.VMEM_SHARED((npad, fp), F32),
            pltpu.SemaphoreType.DMA,
            pltpu.SemaphoreType.DMA,
            pltpu.SemaphoreType.DMA,
            pltpu.SemaphoreType.DMA,
        ],
    )
    def agg_kernel(src_hbm, dst_hbm, hs_hbm, zeros_hbm, out_hbm,
                   src_v, dst_v, rows0, rows1, bounce_v, acc_sh,
                   gs0, gs1, ss0, ss1):
        c = lax.axis_index("c")
        s = lax.axis_index("s")
        wid = c * NS + s
        pltpu.sync_copy(src_hbm.at[wid], src_v)
        pltpu.sync_copy(dst_hbm.at[wid], dst_v)

        for kk in range(nbp):
            slk = pl.ds(s * rpt + kk * rb, rb)

            @pl.when(c == 0)
            def _():
                pltpu.sync_copy(hs_hbm.at[slk], bounce_v)

            @pl.when(c != 0)
            def _():
                pltpu.sync_copy(zeros_hbm.at[slk], bounce_v)

            pltpu.sync_copy(bounce_v, acc_sh.at[slk])
        plsc.subcore_barrier()

        rows = (rows0, rows1)
        gsem = (gs0, gs1)
        ssem = (ss0, ss1)

        def gather(gg, j):
            return pltpu.make_async_copy(hs_hbm.at[src_v.at[gg]], rows[j], gsem[j])

        def scat(gg, j):
            return pltpu.make_async_copy(rows[j], acc_sh.at[dst_v.at[gg]], ssem[j])

        gather(0, 0).start()
        gather(1, 1).start()
        gather(0, 0).wait()
        scat(0, 0).start(add=True)
        gather(1, 1).wait()
        scat(1, 1).start(add=True)

        def body(i, carry):
            g0 = 2 * i
            scat(g0 - 2, 0).wait()
            gather(g0, 0).start()
            scat(g0 - 1, 1).wait()
            gather(g0 + 1, 1).start()
            gather(g0, 0).wait()
            scat(g0, 0).start(add=True)
            gather(g0 + 1, 1).wait()
            scat(g0 + 1, 1).start(add=True)
            return carry

        lax.fori_loop(1, g // 2, body, 0)
        scat(g - 2, 0).wait()
        scat(g - 1, 1).wait()
        plsc.subcore_barrier()
        for kk in range(nbp):
            slk = pl.ds(s * rpt + kk * rb, rb)
            pltpu.sync_copy(acc_sh.at[slk], bounce_v)
            pltpu.sync_copy(bounce_v, out_hbm.at[c, slk])

    return agg_kernel


_RB = 2000


def _tc_first(d0, d1, x, w0):
    n, fin = x.shape
    fout = w0.shape[1]

    def body(d0_r, d1_r, x_r, w_r, dinv_r, hs_r):
        dinv = 1.0 / jnp.sqrt(1.0 + d0_r[...] + d1_r[...])
        h = jnp.dot(x_r[...], w_r[...], preferred_element_type=F32)
        dinv_r[...] = dinv
        hs_r[...] = h * dinv

    return pl.pallas_call(
        body,
        grid=(n // _RB,),
        in_specs=[
            pl.BlockSpec((_RB, 1), lambda i: (i, 0)),
            pl.BlockSpec((_RB, 1), lambda i: (i, 0)),
            pl.BlockSpec((_RB, fin), lambda i: (i, 0)),
            pl.BlockSpec((fin, fout), lambda i: (0, 0)),
        ],
        out_specs=[
            pl.BlockSpec((_RB, 1), lambda i: (i, 0)),
            pl.BlockSpec((_RB, fout), lambda i: (i, 0)),
        ],
        out_shape=[
            jax.ShapeDtypeStruct((n, 1), F32),
            jax.ShapeDtypeStruct((n, fout), F32),
        ],
    )(d0, d1, x, w0)


def _tc_mid(agg, dinv, bvec, scale, beta, w):
    _, n, fin = agg.shape
    fout = w.shape[1]

    def body(agg_r, dinv_r, b_r, s_r, bt_r, w_r, hs_r):
        dinv = dinv_r[...]
        y = (agg_r[0] + agg_r[1]) * dinv + b_r[...]
        act = jnp.where(y > 0, y, jnp.exp(y) - 1.0)
        z = act * s_r[...] + bt_r[...]
        h = jnp.dot(z, w_r[...], preferred_element_type=F32)
        hs_r[...] = h * dinv

    return pl.pallas_call(
        body,
        grid=(n // _RB,),
        in_specs=[
            pl.BlockSpec((2, _RB, fin), lambda i: (0, i, 0)),
            pl.BlockSpec((_RB, 1), lambda i: (i, 0)),
            pl.BlockSpec((1, fin), lambda i: (0, 0)),
            pl.BlockSpec((1, fin), lambda i: (0, 0)),
            pl.BlockSpec((1, fin), lambda i: (0, 0)),
            pl.BlockSpec((fin, fout), lambda i: (0, 0)),
        ],
        out_specs=pl.BlockSpec((_RB, fout), lambda i: (i, 0)),
        out_shape=jax.ShapeDtypeStruct((n, fout), F32),
    )(agg, dinv, bvec, scale, beta, w)


def _tc_final(agg, dinv, bvec):
    _, n, fin = agg.shape

    def body(agg_r, dinv_r, b_r, y_r):
        y_r[...] = (agg_r[0] + agg_r[1]) * dinv_r[...] + b_r[...]

    return pl.pallas_call(
        body,
        grid=(n // _RB,),
        in_specs=[
            pl.BlockSpec((2, _RB, fin), lambda i: (0, i, 0)),
            pl.BlockSpec((_RB, 1), lambda i: (i, 0)),
            pl.BlockSpec((1, fin), lambda i: (0, 0)),
        ],
        out_specs=pl.BlockSpec((_RB, fin), lambda i: (i, 0)),
        out_shape=jax.ShapeDtypeStruct((n, fin), F32),
    )(agg, dinv, bvec)


def kernel(x, edge_index, Ws, bs, gammas, betas):
    n = x.shape[0]
    e = edge_index.shape[1]
    nl = len(Ws)
    dims = [x.shape[1]] + [w.shape[1] for w in Ws]
    fps = [_pad16(d) for d in dims]

    t = math.ceil(e / (NC * NS * CH))
    t = ((t + 7) // 8) * 8
    e_pad = NC * NS * t * CH

    src = jnp.concatenate(
        [edge_index[0], jnp.zeros((e_pad - e,), edge_index.dtype)]
    ).reshape(NC * NS, t * CH)
    dst = jnp.concatenate(
        [edge_index[1], jnp.full((e_pad - e,), n, edge_index.dtype)]
    ).reshape(NC * NS, t * CH)

    npad = n + 8

    def grouped(a, kch):
        return a.reshape(NC * NS, (t * CH) // kch, kch)

    kch_deg, _ = _pick_group(t, LANES, LANES, npad, 1, 1, (n // NS) * LANES)
    degp = _make_sc_degree(n, t)(
        grouped(dst, kch_deg), jnp.zeros((n, LANES), F32),
        jnp.ones((kch_deg, LANES), F32))
    d0 = degp[0, :, 0:1]
    d1 = degp[1, :, 0:1]

    xp = jnp.pad(x, ((0, 0), (0, fps[0] - dims[0])))
    Wp = [jnp.pad(Ws[i], ((0, fps[i] - dims[i]), (0, fps[i + 1] - dims[i + 1])))
          for i in range(nl)]
    bp = [jnp.pad(bs[i], (0, fps[i + 1] - dims[i + 1])).reshape(1, -1)
          for i in range(nl)]
    inv_bn = 1.0 / np.sqrt(1.0 + 1e-5)
    scalep = [(jnp.pad(gammas[i], (0, fps[i + 1] - dims[i + 1])) * inv_bn
               ).reshape(1, -1) for i in range(nl - 1)]
    betap = [jnp.pad(betas[i], (0, fps[i + 1] - dims[i + 1])).reshape(1, -1)
             for i in range(nl - 1)]

    dinv, hs = _tc_first(d0, d1, xp, Wp[0])

    for i in range(nl - 1):
        fp = fps[i + 1]
        _, kch, _ = _agg_bounce_rows(t, fp, npad)
        agg = _make_sc_agg(n, t, fp)(
            grouped(src, kch), grouped(dst, kch), hs, jnp.zeros((n, fp), F32))
        hs = _tc_mid(agg, dinv, bp[i], scalep[i], betap[i], Wp[i + 1])

    fp = fps[nl]
    _, kch, _ = _agg_bounce_rows(t, fp, npad)
    agg = _make_sc_agg(n, t, fp)(
        grouped(src, kch), grouped(dst, kch), hs, jnp.zeros((n, fp), F32))
    y = _tc_final(agg, dinv, bp[nl - 1])
    return y[:, :dims[nl]]


# speedup vs baseline: 1.8525x; 1.8525x over previous
import functools
import math

import jax
import jax.numpy as jnp
import numpy as np
from jax import lax
from jax.experimental import pallas as pl
from jax.experimental.pallas import tpu as pltpu
from jax.experimental.pallas import tpu_sc as plsc

F32 = jnp.float32

NC = 2
NS = 16
LANES = 16
CH = 128


def _pad16(d: int) -> int:
    return ((d + 15) // 16) * 16


def _mesh():
    return plsc.VectorSubcoreMesh(core_axis_name="c", subcore_axis_name="s")


_SPMEM_WORDS = 2097151


def _pick_group(t, acc_fp, row_fp, npad, n_idx, n_rows, fixed_words):
    budget = (_SPMEM_WORDS - npad * acc_fp) // NS - 64
    epw = t * CH
    for kch in (2560, 1280, 640, 512, 320, 256, 128):
        if epw % kch:
            continue
        g = epw // kch
        if g < 2 or g % 2:
            continue
        need = n_rows * kch * row_fp + n_idx * epw + fixed_words
        if need <= budget:
            return kch, g
    raise ValueError("no feasible group size")


def _make_sc_degree(n, t):
    npad = n + 8
    rpt = n // NS
    kch, g = _pick_group(t, LANES, LANES, npad, 1, 1, rpt * LANES)

    @functools.partial(
        pl.kernel,
        out_type=jax.ShapeDtypeStruct((NC, n, LANES), F32),
        mesh=_mesh(),
        compiler_params=pltpu.CompilerParams(use_tc_tiling_on_sc=False),
        scratch_types=[
            pltpu.VMEM((g, kch), jnp.int32),
            pltpu.VMEM((kch, LANES), F32),
            pltpu.VMEM((rpt, LANES), F32),
            pltpu.VMEM_SHARED((npad, LANES), F32),
            pltpu.SemaphoreType.DMA,
        ],
    )
    def deg_kernel(dst_hbm, zeros_hbm, ones_hbm, out_hbm,
                   dst_v, ones_v, bounce_v, acc_sh, sem):
        c = lax.axis_index("c")
        s = lax.axis_index("s")
        wid = c * NS + s
        pltpu.sync_copy(dst_hbm.at[wid], dst_v)
        pltpu.sync_copy(ones_hbm, ones_v)
        sl = pl.ds(s * rpt, rpt)
        pltpu.sync_copy(zeros_hbm.at[sl], bounce_v)
        pltpu.sync_copy(bounce_v, acc_sh.at[sl])
        plsc.subcore_barrier()

        def fire(gg, carry):
            pltpu.make_async_copy(ones_v, acc_sh.at[dst_v.at[gg]], sem).start(
                add=True)
            return carry

        def drain(gg, carry):
            pltpu.make_async_copy(ones_v, acc_sh.at[dst_v.at[gg]], sem).wait()
            return carry

        lax.fori_loop(0, g, fire, 0)
        lax.fori_loop(0, g, drain, 0)
        plsc.subcore_barrier()
        pltpu.sync_copy(acc_sh.at[sl], bounce_v)
        pltpu.sync_copy(bounce_v, out_hbm.at[c, sl])

    return deg_kernel


def _agg_config(t, fp, npad):
    rb = 125
    budget = (_SPMEM_WORDS - 2 * npad * fp) // NS - 64
    epw = t * CH
    for kch in (1280, 640, 512, 320, 256, 128):
        if epw % kch:
            continue
        g = epw // kch
        if g < 2 or g % 2:
            continue
        need = 2 * kch * fp + 4 * kch + rb * fp
        if need <= budget:
            return rb, kch, g
    raise ValueError("no feasible group size")


def _make_sc_agg(n, t, fp):
    npad = n + 8
    rpt = n // NS
    rb, kch, g = _agg_config(t, fp, npad)
    nbp = rpt // rb

    @functools.partial(
        pl.kernel,
        out_type=jax.ShapeDtypeStruct((NC, n, fp), F32),
        mesh=_mesh(),
        compiler_params=pltpu.CompilerParams(use_tc_tiling_on_sc=False),
        scratch_types=[
            pltpu.VMEM((2, kch), jnp.int32),
            pltpu.VMEM((2, kch), jnp.int32),
            pltpu.VMEM((kch, fp), F32),
            pltpu.VMEM((kch, fp), F32),
            pltpu.VMEM((rb, fp), F32),
            pltpu.VMEM_SHARED((npad, fp), F32),
            pltpu.VMEM_SHARED((npad, fp), F32),
            pltpu.SemaphoreType.DMA,
            pltpu.SemaphoreType.DMA,
            pltpu.SemaphoreType.DMA,
            pltpu.SemaphoreType.DMA,
            pltpu.SemaphoreType.DMA,
            pltpu.SemaphoreType.DMA,
        ],
    )
    def agg_kernel(idx_hbm, hs_hbm, zeros_hbm, out_hbm,
                   idx0, idx1, rows0, rows1, bounce_v, table_sh, acc_sh,
                   is0, is1, gs0, gs1, ss0, ss1):
        c = lax.axis_index("c")
        s = lax.axis_index("s")
        wid = c * NS + s

        for kk in range(nbp):
            slk = pl.ds(s * rpt + kk * rb, rb)
            pltpu.sync_copy(hs_hbm.at[slk], bounce_v)
            pltpu.sync_copy(bounce_v, table_sh.at[slk])

            @pl.when(c == 0)
            def _():
                pltpu.sync_copy(bounce_v, acc_sh.at[slk])

        @pl.when(c != 0)
        def _():
            for kk in range(nbp):
                slk = pl.ds(s * rpt + kk * rb, rb)
                pltpu.sync_copy(zeros_hbm.at[slk], bounce_v)
                pltpu.sync_copy(bounce_v, acc_sh.at[slk])

        plsc.subcore_barrier()

        idx = (idx0, idx1)
        rows = (rows0, rows1)
        isem = (is0, is1)
        gsem = (gs0, gs1)
        ssem = (ss0, ss1)

        def idxc(gg, j):
            return pltpu.make_async_copy(idx_hbm.at[wid, gg], idx[j], isem[j])

        def gather(gg, j):
            return pltpu.make_async_copy(
                table_sh.at[idx[j].at[0]], rows[j], gsem[j])

        def scat(gg, j):
            return pltpu.make_async_copy(
                rows[j], acc_sh.at[idx[j].at[1]], ssem[j])

        idxc(0, 0).start()
        idxc(1, 1).start()
        idxc(0, 0).wait()
        gather(0, 0).start()
        idxc(1, 1).wait()
        gather(1, 1).start()
        gather(0, 0).wait()
        scat(0, 0).start(add=True)
        gather(1, 1).wait()
        scat(1, 1).start(add=True)

        def body(i, carry):
            g0 = 2 * i
            scat(g0 - 2, 0).wait()
            idxc(g0, 0).start()
            scat(g0 - 1, 1).wait()
            idxc(g0 + 1, 1).start()
            idxc(g0, 0).wait()
            gather(g0, 0).start()
            idxc(g0 + 1, 1).wait()
            gather(g0 + 1, 1).start()
            gather(g0, 0).wait()
            scat(g0, 0).start(add=True)
            gather(g0 + 1, 1).wait()
            scat(g0 + 1, 1).start(add=True)
            return carry

        lax.fori_loop(1, g // 2, body, 0)
        scat(g - 2, 0).wait()
        scat(g - 1, 1).wait()
        plsc.subcore_barrier()
        for kk in range(nbp):
            slk = pl.ds(s * rpt + kk * rb, rb)
            pltpu.sync_copy(acc_sh.at[slk], bounce_v)
            pltpu.sync_copy(bounce_v, out_hbm.at[c, slk])

    return agg_kernel


_RB = 2000


def _tc_first(d0, d1, x, w0):
    n, fin = x.shape
    fout = w0.shape[1]

    def body(d0_r, d1_r, x_r, w_r, dinv_r, hs_r):
        dinv = 1.0 / jnp.sqrt(1.0 + d0_r[...] + d1_r[...])
        h = jnp.dot(x_r[...], w_r[...], preferred_element_type=F32)
        dinv_r[...] = dinv
        hs_r[...] = h * dinv

    return pl.pallas_call(
        body,
        grid=(n // _RB,),
        in_specs=[
            pl.BlockSpec((_RB, 1), lambda i: (i, 0)),
            pl.BlockSpec((_RB, 1), lambda i: (i, 0)),
            pl.BlockSpec((_RB, fin), lambda i: (i, 0)),
            pl.BlockSpec((fin, fout), lambda i: (0, 0)),
        ],
        out_specs=[
            pl.BlockSpec((_RB, 1), lambda i: (i, 0)),
            pl.BlockSpec((_RB, fout), lambda i: (i, 0)),
        ],
        out_shape=[
            jax.ShapeDtypeStruct((n, 1), F32),
            jax.ShapeDtypeStruct((n, fout), F32),
        ],
    )(d0, d1, x, w0)


def _tc_mid(agg, dinv, bvec, scale, beta, w):
    _, n, fin = agg.shape
    fout = w.shape[1]

    def body(agg_r, dinv_r, b_r, s_r, bt_r, w_r, hs_r):
        dinv = dinv_r[...]
        y = (agg_r[0] + agg_r[1]) * dinv + b_r[...]
        act = jnp.where(y > 0, y, jnp.exp(y) - 1.0)
        z = act * s_r[...] + bt_r[...]
        h = jnp.dot(z, w_r[...], preferred_element_type=F32)
        hs_r[...] = h * dinv

    return pl.pallas_call(
        body,
        grid=(n // _RB,),
        in_specs=[
            pl.BlockSpec((2, _RB, fin), lambda i: (0, i, 0)),
            pl.BlockSpec((_RB, 1), lambda i: (i, 0)),
            pl.BlockSpec((1, fin), lambda i: (0, 0)),
            pl.BlockSpec((1, fin), lambda i: (0, 0)),
            pl.BlockSpec((1, fin), lambda i: (0, 0)),
            pl.BlockSpec((fin, fout), lambda i: (0, 0)),
        ],
        out_specs=pl.BlockSpec((_RB, fout), lambda i: (i, 0)),
        out_shape=jax.ShapeDtypeStruct((n, fout), F32),
    )(agg, dinv, bvec, scale, beta, w)


def _tc_final(agg, dinv, bvec):
    _, n, fin = agg.shape

    def body(agg_r, dinv_r, b_r, y_r):
        y_r[...] = (agg_r[0] + agg_r[1]) * dinv_r[...] + b_r[...]

    return pl.pallas_call(
        body,
        grid=(n // _RB,),
        in_specs=[
            pl.BlockSpec((2, _RB, fin), lambda i: (0, i, 0)),
            pl.BlockSpec((_RB, 1), lambda i: (i, 0)),
            pl.BlockSpec((1, fin), lambda i: (0, 0)),
        ],
        out_specs=pl.BlockSpec((_RB, fin), lambda i: (i, 0)),
        out_shape=jax.ShapeDtypeStruct((n, fin), F32),
    )(agg, dinv, bvec)


def kernel(x, edge_index, Ws, bs, gammas, betas):
    n = x.shape[0]
    e = edge_index.shape[1]
    nl = len(Ws)
    dims = [x.shape[1]] + [w.shape[1] for w in Ws]
    fps = [_pad16(d) for d in dims]

    t = math.ceil(e / (NC * NS * CH))
    t = ((t + 7) // 8) * 8
    e_pad = NC * NS * t * CH

    src = jnp.concatenate(
        [edge_index[0], jnp.zeros((e_pad - e,), edge_index.dtype)]
    ).reshape(NC * NS, t * CH)
    dst = jnp.concatenate(
        [edge_index[1], jnp.full((e_pad - e,), n, edge_index.dtype)]
    ).reshape(NC * NS, t * CH)

    npad = n + 8

    def grouped(a, kch):
        return a.reshape(NC * NS, (t * CH) // kch, kch)

    def interleaved(kch):
        gg = (t * CH) // kch
        return jnp.stack(
            [src.reshape(NC * NS, gg, kch), dst.reshape(NC * NS, gg, kch)],
            axis=2)

    kch_deg, _ = _pick_group(t, LANES, LANES, npad, 1, 1, (n // NS) * LANES)
    degp = _make_sc_degree(n, t)(
        grouped(dst, kch_deg), jnp.zeros((n, LANES), F32),
        jnp.ones((kch_deg, LANES), F32))
    d0 = degp[0, :, 0:1]
    d1 = degp[1, :, 0:1]

    xp = jnp.pad(x, ((0, 0), (0, fps[0] - dims[0])))
    Wp = [jnp.pad(Ws[i], ((0, fps[i] - dims[i]), (0, fps[i + 1] - dims[i + 1])))
          for i in range(nl)]
    bp = [jnp.pad(bs[i], (0, fps[i + 1] - dims[i + 1])).reshape(1, -1)
          for i in range(nl)]
    inv_bn = 1.0 / np.sqrt(1.0 + 1e-5)
    scalep = [(jnp.pad(gammas[i], (0, fps[i + 1] - dims[i + 1])) * inv_bn
               ).reshape(1, -1) for i in range(nl - 1)]
    betap = [jnp.pad(betas[i], (0, fps[i + 1] - dims[i + 1])).reshape(1, -1)
             for i in range(nl - 1)]

    dinv, hs = _tc_first(d0, d1, xp, Wp[0])

    for i in range(nl - 1):
        fp = fps[i + 1]
        _, kch, _ = _agg_config(t, fp, npad)
        agg = _make_sc_agg(n, t, fp)(
            interleaved(kch), hs, jnp.zeros((n, fp), F32))
        hs = _tc_mid(agg, dinv, bp[i], scalep[i], betap[i], Wp[i + 1])

    fp = fps[nl]
    _, kch, _ = _agg_config(t, fp, npad)
    agg = _make_sc_agg(n, t, fp)(
        interleaved(kch), hs, jnp.zeros((n, fp), F32))
    y = _tc_final(agg, dinv, bp[nl - 1])
    return y[:, :dims[nl]]


# speedup vs baseline: 2.0520x; 1.1077x over previous
import functools
import math

import jax
import jax.numpy as jnp
import numpy as np
from jax import lax
from jax.experimental import pallas as pl
from jax.experimental.pallas import tpu as pltpu
from jax.experimental.pallas import tpu_sc as plsc

F32 = jnp.float32

NC = 2
NS = 16
LANES = 16
CH = 128


def _pad8(d: int) -> int:
    return ((d + 7) // 8) * 8


def _mesh():
    return plsc.VectorSubcoreMesh(core_axis_name="c", subcore_axis_name="s")


_SPMEM_WORDS = 2097151


def _pick_group(t, acc_fp, row_fp, npad, n_idx, n_rows, fixed_words):
    budget = (_SPMEM_WORDS - npad * acc_fp) // NS - 64
    epw = t * CH
    for kch in (2560, 1280, 640, 512, 320, 256, 128):
        if epw % kch:
            continue
        g = epw // kch
        if g < 2 or g % 2:
            continue
        need = n_rows * kch * row_fp + n_idx * epw + fixed_words
        if need <= budget:
            return kch, g
    raise ValueError("no feasible group size")


def _make_sc_degree(n, t):
    npad = n + 8
    rpt = n // NS
    kch, g = _pick_group(t, LANES, LANES, npad, 1, 1, rpt * LANES)

    @functools.partial(
        pl.kernel,
        out_type=jax.ShapeDtypeStruct((NC, n, LANES), F32),
        mesh=_mesh(),
        compiler_params=pltpu.CompilerParams(use_tc_tiling_on_sc=False),
        scratch_types=[
            pltpu.VMEM((g, kch), jnp.int32),
            pltpu.VMEM((kch, LANES), F32),
            pltpu.VMEM_SHARED((npad, LANES), F32),
            pltpu.SemaphoreType.DMA,
        ],
    )
    def deg_kernel(dst_hbm, zeros_hbm, ones_hbm, out_hbm,
                   dst_v, ones_v, acc_sh, sem):
        c = lax.axis_index("c")
        s = lax.axis_index("s")
        wid = c * NS + s
        pltpu.sync_copy(dst_hbm.at[wid], dst_v)
        pltpu.sync_copy(ones_hbm, ones_v)
        sl = pl.ds(s * rpt, rpt)
        pltpu.sync_copy(zeros_hbm.at[sl], acc_sh.at[sl])
        plsc.subcore_barrier()

        def fire(gg, carry):
            pltpu.make_async_copy(ones_v, acc_sh.at[dst_v.at[gg]], sem).start(
                add=True)
            return carry

        def drain(gg, carry):
            pltpu.make_async_copy(ones_v, acc_sh.at[dst_v.at[gg]], sem).wait()
            return carry

        lax.fori_loop(0, g, fire, 0)
        lax.fori_loop(0, g, drain, 0)
        plsc.subcore_barrier()
        pltpu.sync_copy(acc_sh.at[sl], out_hbm.at[c, sl])

    return deg_kernel


def _agg_config(t, fp, npad):
    budget = (_SPMEM_WORDS - 2 * npad * fp) // NS - 64
    epw = t * CH
    for kch in (1280, 640, 512, 320, 256, 128):
        if epw % kch:
            continue
        g = epw // kch
        if g < 2 or g % 2:
            continue
        need = 2 * kch * fp + 4 * kch
        if need <= budget:
            return kch, g
    raise ValueError("no feasible group size")


def _make_sc_agg(n, t, fp):
    npad = n + 8
    rpt = n // NS
    kch, g = _agg_config(t, fp, npad)

    @functools.partial(
        pl.kernel,
        out_type=jax.ShapeDtypeStruct((NC, n, fp), F32),
        mesh=_mesh(),
        compiler_params=pltpu.CompilerParams(use_tc_tiling_on_sc=False),
        scratch_types=[
            pltpu.VMEM((2, kch), jnp.int32),
            pltpu.VMEM((2, kch), jnp.int32),
            pltpu.VMEM((kch, fp), F32),
            pltpu.VMEM((kch, fp), F32),
            pltpu.VMEM_SHARED((npad, fp), F32),
            pltpu.VMEM_SHARED((npad, fp), F32),
            pltpu.SemaphoreType.DMA,
            pltpu.SemaphoreType.DMA,
            pltpu.SemaphoreType.DMA,
            pltpu.SemaphoreType.DMA,
            pltpu.SemaphoreType.DMA,
            pltpu.SemaphoreType.DMA,
        ],
    )
    def agg_kernel(idx_hbm, hs_hbm, zeros_hbm, out_hbm,
                   idx0, idx1, rows0, rows1, table_sh, acc_sh,
                   is0, is1, gs0, gs1, ss0, ss1):
        c = lax.axis_index("c")
        s = lax.axis_index("s")
        wid = c * NS + s

        sl = pl.ds(s * rpt, rpt)
        pltpu.sync_copy(hs_hbm.at[sl], table_sh.at[sl])

        @pl.when(c == 0)
        def _():
            pltpu.sync_copy(hs_hbm.at[sl], acc_sh.at[sl])

        @pl.when(c != 0)
        def _():
            pltpu.sync_copy(zeros_hbm.at[sl], acc_sh.at[sl])

        plsc.subcore_barrier()

        idx = (idx0, idx1)
        rows = (rows0, rows1)
        isem = (is0, is1)
        gsem = (gs0, gs1)
        ssem = (ss0, ss1)

        def idxc(gg, j):
            return pltpu.make_async_copy(idx_hbm.at[wid, gg], idx[j], isem[j])

        def gather(gg, j):
            return pltpu.make_async_copy(
                table_sh.at[idx[j].at[0]], rows[j], gsem[j])

        def scat(gg, j):
            return pltpu.make_async_copy(
                rows[j], acc_sh.at[idx[j].at[1]], ssem[j])

        idxc(0, 0).start()
        idxc(1, 1).start()
        idxc(0, 0).wait()
        gather(0, 0).start()
        idxc(1, 1).wait()
        gather(1, 1).start()
        gather(0, 0).wait()
        scat(0, 0).start(add=True)
        gather(1, 1).wait()
        scat(1, 1).start(add=True)

        def body(i, carry):
            g0 = 2 * i
            scat(g0 - 2, 0).wait()
            idxc(g0, 0).start()
            scat(g0 - 1, 1).wait()
            idxc(g0 + 1, 1).start()
            idxc(g0, 0).wait()
            gather(g0, 0).start()
            idxc(g0 + 1, 1).wait()
            gather(g0 + 1, 1).start()
            gather(g0, 0).wait()
            scat(g0, 0).start(add=True)
            gather(g0 + 1, 1).wait()
            scat(g0 + 1, 1).start(add=True)
            return carry

        lax.fori_loop(1, g // 2, body, 0)
        scat(g - 2, 0).wait()
        scat(g - 1, 1).wait()
        plsc.subcore_barrier()
        pltpu.sync_copy(acc_sh.at[sl], out_hbm.at[c, sl])

    return agg_kernel


_RB = 2000


def _tc_first(d0, d1, x, w0):
    n, fin = x.shape
    fout = w0.shape[1]

    def body(d0_r, d1_r, x_r, w_r, dinv_r, hs_r):
        dinv = 1.0 / jnp.sqrt(1.0 + d0_r[...] + d1_r[...])
        h = jnp.dot(x_r[...], w_r[...], preferred_element_type=F32)
        dinv_r[...] = dinv
        hs_r[...] = h * dinv

    return pl.pallas_call(
        body,
        grid=(n // _RB,),
        in_specs=[
            pl.BlockSpec((_RB, 1), lambda i: (i, 0)),
            pl.BlockSpec((_RB, 1), lambda i: (i, 0)),
            pl.BlockSpec((_RB, fin), lambda i: (i, 0)),
            pl.BlockSpec((fin, fout), lambda i: (0, 0)),
        ],
        out_specs=[
            pl.BlockSpec((_RB, 1), lambda i: (i, 0)),
            pl.BlockSpec((_RB, fout), lambda i: (i, 0)),
        ],
        out_shape=[
            jax.ShapeDtypeStruct((n, 1), F32),
            jax.ShapeDtypeStruct((n, fout), F32),
        ],
    )(d0, d1, x, w0)


def _tc_mid(agg, dinv, bvec, scale, beta, w):
    _, n, fin = agg.shape
    fout = w.shape[1]

    def body(agg_r, dinv_r, b_r, s_r, bt_r, w_r, hs_r):
        dinv = dinv_r[...]
        y = (agg_r[0] + agg_r[1]) * dinv + b_r[...]
        act = jnp.where(y > 0, y, jnp.exp(y) - 1.0)
        z = act * s_r[...] + bt_r[...]
        h = jnp.dot(z, w_r[...], preferred_element_type=F32)
        hs_r[...] = h * dinv

    return pl.pallas_call(
        body,
        grid=(n // _RB,),
        in_specs=[
            pl.BlockSpec((2, _RB, fin), lambda i: (0, i, 0)),
            pl.BlockSpec((_RB, 1), lambda i: (i, 0)),
            pl.BlockSpec((1, fin), lambda i: (0, 0)),
            pl.BlockSpec((1, fin), lambda i: (0, 0)),
            pl.BlockSpec((1, fin), lambda i: (0, 0)),
            pl.BlockSpec((fin, fout), lambda i: (0, 0)),
        ],
        out_specs=pl.BlockSpec((_RB, fout), lambda i: (i, 0)),
        out_shape=jax.ShapeDtypeStruct((n, fout), F32),
    )(agg, dinv, bvec, scale, beta, w)


def _tc_final(agg, dinv, bvec):
    _, n, fin = agg.shape

    def body(agg_r, dinv_r, b_r, y_r):
        y_r[...] = (agg_r[0] + agg_r[1]) * dinv_r[...] + b_r[...]

    return pl.pallas_call(
        body,
        grid=(n // _RB,),
        in_specs=[
            pl.BlockSpec((2, _RB, fin), lambda i: (0, i, 0)),
            pl.BlockSpec((_RB, 1), lambda i: (i, 0)),
            pl.BlockSpec((1, fin), lambda i: (0, 0)),
        ],
        out_specs=pl.BlockSpec((_RB, fin), lambda i: (i, 0)),
        out_shape=jax.ShapeDtypeStruct((n, fin), F32),
    )(agg, dinv, bvec)


def kernel(x, edge_index, Ws, bs, gammas, betas):
    n = x.shape[0]
    e = edge_index.shape[1]
    nl = len(Ws)
    dims = [x.shape[1]] + [w.shape[1] for w in Ws]
    fps = [_pad8(d) for d in dims]

    t = math.ceil(e / (NC * NS * CH))
    t = ((t + 7) // 8) * 8
    e_pad = NC * NS * t * CH

    src = jnp.concatenate(
        [edge_index[0], jnp.zeros((e_pad - e,), edge_index.dtype)]
    ).reshape(NC * NS, t * CH)
    dst = jnp.concatenate(
        [edge_index[1], jnp.full((e_pad - e,), n, edge_index.dtype)]
    ).reshape(NC * NS, t * CH)

    npad = n + 8

    def grouped(a, kch):
        return a.reshape(NC * NS, (t * CH) // kch, kch)

    def interleaved(kch):
        gg = (t * CH) // kch
        return jnp.stack(
            [src.reshape(NC * NS, gg, kch), dst.reshape(NC * NS, gg, kch)],
            axis=2)

    kch_deg, _ = _pick_group(t, LANES, LANES, npad, 1, 1, (n // NS) * LANES)
    degp = _make_sc_degree(n, t)(
        grouped(dst, kch_deg), jnp.zeros((n, LANES), F32),
        jnp.ones((kch_deg, LANES), F32))
    d0 = degp[0, :, 0:1]
    d1 = degp[1, :, 0:1]

    xp = jnp.pad(x, ((0, 0), (0, fps[0] - dims[0])))
    Wp = [jnp.pad(Ws[i], ((0, fps[i] - dims[i]), (0, fps[i + 1] - dims[i + 1])))
          for i in range(nl)]
    bp = [jnp.pad(bs[i], (0, fps[i + 1] - dims[i + 1])).reshape(1, -1)
          for i in range(nl)]
    inv_bn = 1.0 / np.sqrt(1.0 + 1e-5)
    scalep = [(jnp.pad(gammas[i], (0, fps[i + 1] - dims[i + 1])) * inv_bn
               ).reshape(1, -1) for i in range(nl - 1)]
    betap = [jnp.pad(betas[i], (0, fps[i + 1] - dims[i + 1])).reshape(1, -1)
             for i in range(nl - 1)]

    dinv, hs = _tc_first(d0, d1, xp, Wp[0])

    for i in range(nl - 1):
        fp = fps[i + 1]
        kch, _ = _agg_config(t, fp, npad)
        agg = _make_sc_agg(n, t, fp)(
            interleaved(kch), hs, jnp.zeros((n, fp), F32))
        hs = _tc_mid(agg, dinv, bp[i], scalep[i], betap[i], Wp[i + 1])

    fp = fps[nl]
    kch, _ = _agg_config(t, fp, npad)
    agg = _make_sc_agg(n, t, fp)(
        interleaved(kch), hs, jnp.zeros((n, fp), F32))
    y = _tc_final(agg, dinv, bp[nl - 1])
    return y[:, :dims[nl]]


# speedup vs baseline: 2.0542x; 1.0010x over previous
import functools
import math

import jax
import jax.numpy as jnp
import numpy as np
from jax import lax
from jax.experimental import pallas as pl
from jax.experimental.pallas import tpu as pltpu
from jax.experimental.pallas import tpu_sc as plsc

F32 = jnp.float32

NC = 2
NS = 16
LANES = 16
CH = 128


def _pad8(d: int) -> int:
    return ((d + 7) // 8) * 8


def _mesh():
    return plsc.VectorSubcoreMesh(core_axis_name="c", subcore_axis_name="s")


_SPMEM_WORDS = 2097151


def _pick_group(t, acc_fp, row_fp, npad, n_idx, n_rows, fixed_words):
    budget = (_SPMEM_WORDS - npad * acc_fp) // NS - 64
    epw = t * CH
    for kch in (2560, 1280, 640, 512, 320, 256, 128):
        if epw % kch:
            continue
        g = epw // kch
        if g < 2 or g % 2:
            continue
        need = n_rows * kch * row_fp + n_idx * epw + fixed_words
        if need <= budget:
            return kch, g
    raise ValueError("no feasible group size")


def _make_sc_degree(n, t):
    npad = n + 8
    rpt = n // NS
    kch, g = _pick_group(t, LANES, LANES, npad, 1, 1, rpt * LANES)

    @functools.partial(
        pl.kernel,
        out_type=jax.ShapeDtypeStruct((NC, n, LANES), F32),
        mesh=_mesh(),
        compiler_params=pltpu.CompilerParams(use_tc_tiling_on_sc=False),
        scratch_types=[
            pltpu.VMEM((g, kch), jnp.int32),
            pltpu.VMEM((kch, LANES), F32),
            pltpu.VMEM_SHARED((npad, LANES), F32),
            pltpu.SemaphoreType.DMA,
        ],
    )
    def deg_kernel(dst_hbm, zeros_hbm, ones_hbm, out_hbm,
                   dst_v, ones_v, acc_sh, sem):
        c = lax.axis_index("c")
        s = lax.axis_index("s")
        wid = c * NS + s
        pltpu.sync_copy(dst_hbm.at[wid], dst_v)
        pltpu.sync_copy(ones_hbm, ones_v)
        sl = pl.ds(s * rpt, rpt)
        pltpu.sync_copy(zeros_hbm.at[sl], acc_sh.at[sl])
        plsc.subcore_barrier()

        def fire(gg, carry):
            pltpu.make_async_copy(ones_v, acc_sh.at[dst_v.at[gg]], sem).start(
                add=True)
            return carry

        def drain(gg, carry):
            pltpu.make_async_copy(ones_v, acc_sh.at[dst_v.at[gg]], sem).wait()
            return carry

        lax.fori_loop(0, g, fire, 0)
        lax.fori_loop(0, g, drain, 0)
        plsc.subcore_barrier()
        pltpu.sync_copy(acc_sh.at[sl], out_hbm.at[c, sl])

    return deg_kernel


def _agg_config(t, fp, npad):
    budget = (_SPMEM_WORDS - 2 * npad * fp) // NS - 64
    epw = t * CH
    for kch in (1280, 640, 512, 320, 256, 128):
        if epw % kch:
            continue
        g = epw // kch
        if g < 2 or g % 2:
            continue
        need = 2 * kch * fp + 4 * kch
        if need <= budget:
            return kch, g
    raise ValueError("no feasible group size")


def _make_sc_agg(n, t, fp):
    npad = n + 8
    rpt = n // NS
    kch, g = _agg_config(t, fp, npad)

    @functools.partial(
        pl.kernel,
        out_type=jax.ShapeDtypeStruct((NC, n, fp), F32),
        mesh=_mesh(),
        compiler_params=pltpu.CompilerParams(use_tc_tiling_on_sc=False),
        scratch_types=[
            pltpu.VMEM((2, kch), jnp.int32),
            pltpu.VMEM((2, kch), jnp.int32),
            pltpu.VMEM((kch, fp), F32),
            pltpu.VMEM((kch, fp), F32),
            pltpu.VMEM_SHARED((npad, fp), F32),
            pltpu.VMEM_SHARED((npad, fp), F32),
            pltpu.SemaphoreType.DMA,
            pltpu.SemaphoreType.DMA,
            pltpu.SemaphoreType.DMA,
            pltpu.SemaphoreType.DMA,
            pltpu.SemaphoreType.DMA,
            pltpu.SemaphoreType.DMA,
        ],
    )
    def agg_kernel(idx_hbm, hs_hbm, zeros_hbm, out_hbm,
                   idx0, idx1, rows0, rows1, table_sh, acc_sh,
                   is0, is1, gs0, gs1, ss0, ss1):
        c = lax.axis_index("c")
        s = lax.axis_index("s")
        wid = c * NS + s

        sl = pl.ds(s * rpt, rpt)
        pltpu.sync_copy(hs_hbm.at[sl], table_sh.at[sl])

        @pl.when(c == 0)
        def _():
            pltpu.sync_copy(hs_hbm.at[sl], acc_sh.at[sl])

        @pl.when(c != 0)
        def _():
            pltpu.sync_copy(zeros_hbm.at[sl], acc_sh.at[sl])

        plsc.subcore_barrier()

        idx = (idx0, idx1)
        rows = (rows0, rows1)
        isem = (is0, is1)
        gsem = (gs0, gs1)
        ssem = (ss0, ss1)

        def idxc(gg, j):
            return pltpu.make_async_copy(idx_hbm.at[wid, gg], idx[j], isem[j])

        def gather(gg, j):
            return pltpu.make_async_copy(
                table_sh.at[idx[j].at[0]], rows[j], gsem[j])

        def scat(gg, j):
            return pltpu.make_async_copy(
                rows[j], acc_sh.at[idx[j].at[1]], ssem[j])

        idxc(0, 0).start()
        idxc(1, 1).start()
        idxc(0, 0).wait()
        gather(0, 0).start()
        idxc(1, 1).wait()
        gather(1, 1).start()
        gather(0, 0).wait()
        scat(0, 0).start(add=True)
        gather(1, 1).wait()
        scat(1, 1).start(add=True)

        def body(i, carry):
            g0 = 2 * i
            scat(g0 - 2, 0).wait()
            idxc(g0, 0).start()
            scat(g0 - 1, 1).wait()
            idxc(g0 + 1, 1).start()
            idxc(g0, 0).wait()
            gather(g0, 0).start()
            idxc(g0 + 1, 1).wait()
            gather(g0 + 1, 1).start()
            gather(g0, 0).wait()
            scat(g0, 0).start(add=True)
            gather(g0 + 1, 1).wait()
            scat(g0 + 1, 1).start(add=True)
            return carry

        lax.fori_loop(1, g // 2, body, 0)
        scat(g - 2, 0).wait()
        scat(g - 1, 1).wait()
        plsc.subcore_barrier()
        pltpu.sync_copy(acc_sh.at[sl], out_hbm.at[c, sl])

    return agg_kernel


_RB = 2000


def _tc_mm0(x, w0):
    n, fin = x.shape
    fout = w0.shape[1]

    def body(x_r, w_r, h_r):
        h_r[...] = jnp.dot(x_r[...], w_r[...], preferred_element_type=F32)

    return pl.pallas_call(
        body,
        grid=(n // _RB,),
        in_specs=[
            pl.BlockSpec((_RB, fin), lambda i: (i, 0)),
            pl.BlockSpec((fin, fout), lambda i: (0, 0)),
        ],
        out_specs=pl.BlockSpec((_RB, fout), lambda i: (i, 0)),
        out_shape=jax.ShapeDtypeStruct((n, fout), F32),
    )(x, w0)


def _tc_scale(d0, d1, h):
    n, fout = h.shape

    def body(d0_r, d1_r, h_r, dinv_r, hs_r):
        dinv = 1.0 / jnp.sqrt(1.0 + d0_r[...] + d1_r[...])
        dinv_r[...] = dinv
        hs_r[...] = h_r[...] * dinv

    return pl.pallas_call(
        body,
        grid=(n // _RB,),
        in_specs=[
            pl.BlockSpec((_RB, 1), lambda i: (i, 0)),
            pl.BlockSpec((_RB, 1), lambda i: (i, 0)),
            pl.BlockSpec((_RB, fout), lambda i: (i, 0)),
        ],
        out_specs=[
            pl.BlockSpec((_RB, 1), lambda i: (i, 0)),
            pl.BlockSpec((_RB, fout), lambda i: (i, 0)),
        ],
        out_shape=[
            jax.ShapeDtypeStruct((n, 1), F32),
            jax.ShapeDtypeStruct((n, fout), F32),
        ],
    )(d0, d1, h)


def _tc_mid(agg, dinv, bvec, scale, beta, w):
    _, n, fin = agg.shape
    fout = w.shape[1]

    def body(agg_r, dinv_r, b_r, s_r, bt_r, w_r, hs_r):
        dinv = dinv_r[...]
        y = (agg_r[0] + agg_r[1]) * dinv + b_r[...]
        act = jnp.where(y > 0, y, jnp.exp(y) - 1.0)
        z = act * s_r[...] + bt_r[...]
        h = jnp.dot(z, w_r[...], preferred_element_type=F32)
        hs_r[...] = h * dinv

    return pl.pallas_call(
        body,
        grid=(n // _RB,),
        in_specs=[
            pl.BlockSpec((2, _RB, fin), lambda i: (0, i, 0)),
            pl.BlockSpec((_RB, 1), lambda i: (i, 0)),
            pl.BlockSpec((1, fin), lambda i: (0, 0)),
            pl.BlockSpec((1, fin), lambda i: (0, 0)),
            pl.BlockSpec((1, fin), lambda i: (0, 0)),
            pl.BlockSpec((fin, fout), lambda i: (0, 0)),
        ],
        out_specs=pl.BlockSpec((_RB, fout), lambda i: (i, 0)),
        out_shape=jax.ShapeDtypeStruct((n, fout), F32),
    )(agg, dinv, bvec, scale, beta, w)


def _tc_final(agg, dinv, bvec):
    _, n, fin = agg.shape

    def body(agg_r, dinv_r, b_r, y_r):
        y_r[...] = (agg_r[0] + agg_r[1]) * dinv_r[...] + b_r[...]

    return pl.pallas_call(
        body,
        grid=(n // _RB,),
        in_specs=[
            pl.BlockSpec((2, _RB, fin), lambda i: (0, i, 0)),
            pl.BlockSpec((_RB, 1), lambda i: (i, 0)),
            pl.BlockSpec((1, fin), lambda i: (0, 0)),
        ],
        out_specs=pl.BlockSpec((_RB, fin), lambda i: (i, 0)),
        out_shape=jax.ShapeDtypeStruct((n, fin), F32),
    )(agg, dinv, bvec)


def kernel(x, edge_index, Ws, bs, gammas, betas):
    n = x.shape[0]
    e = edge_index.shape[1]
    nl = len(Ws)
    dims = [x.shape[1]] + [w.shape[1] for w in Ws]
    fps = [_pad8(d) for d in dims]

    t = math.ceil(e / (NC * NS * CH))
    t = ((t + 7) // 8) * 8
    e_pad = NC * NS * t * CH

    src = jnp.concatenate(
        [edge_index[0], jnp.zeros((e_pad - e,), edge_index.dtype)]
    ).reshape(NC * NS, t * CH)
    dst = jnp.concatenate(
        [edge_index[1], jnp.full((e_pad - e,), n, edge_index.dtype)]
    ).reshape(NC * NS, t * CH)

    npad = n + 8

    def grouped(a, kch):
        return a.reshape(NC * NS, (t * CH) // kch, kch)

    def interleaved(kch):
        gg = (t * CH) // kch
        return jnp.stack(
            [src.reshape(NC * NS, gg, kch), dst.reshape(NC * NS, gg, kch)],
            axis=2)

    kch_deg, _ = _pick_group(t, LANES, LANES, npad, 1, 1, (n // NS) * LANES)
    degp = _make_sc_degree(n, t)(
        grouped(dst, kch_deg), jnp.zeros((n, LANES), F32),
        jnp.ones((kch_deg, LANES), F32))
    d0 = degp[0, :, 0:1]
    d1 = degp[1, :, 0:1]

    xp = jnp.pad(x, ((0, 0), (0, fps[0] - dims[0])))
    Wp = [jnp.pad(Ws[i], ((0, fps[i] - dims[i]), (0, fps[i + 1] - dims[i + 1])))
          for i in range(nl)]
    bp = [jnp.pad(bs[i], (0, fps[i + 1] - dims[i + 1])).reshape(1, -1)
          for i in range(nl)]
    inv_bn = 1.0 / np.sqrt(1.0 + 1e-5)
    scalep = [(jnp.pad(gammas[i], (0, fps[i + 1] - dims[i + 1])) * inv_bn
               ).reshape(1, -1) for i in range(nl - 1)]
    betap = [jnp.pad(betas[i], (0, fps[i + 1] - dims[i + 1])).reshape(1, -1)
             for i in range(nl - 1)]

    h0 = _tc_mm0(xp, Wp[0])
    dinv, hs = _tc_scale(d0, d1, h0)

    for i in range(nl - 1):
        fp = fps[i + 1]
        kch, _ = _agg_config(t, fp, npad)
        agg = _make_sc_agg(n, t, fp)(
            interleaved(kch), hs, jnp.zeros((n, fp), F32))
        hs = _tc_mid(agg, dinv, bp[i], scalep[i], betap[i], Wp[i + 1])

    fp = fps[nl]
    kch, _ = _agg_config(t, fp, npad)
    agg = _make_sc_agg(n, t, fp)(
        interleaved(kch), hs, jnp.zeros((n, fp), F32))
    y = _tc_final(agg, dinv, bp[nl - 1])
    return y[:, :dims[nl]]


# speedup vs baseline: 2.4757x; 1.2052x over previous
import functools
import math

import jax
import jax.numpy as jnp
import numpy as np
from jax import lax
from jax.experimental import pallas as pl
from jax.experimental.pallas import tpu as pltpu
from jax.experimental.pallas import tpu_sc as plsc

F32 = jnp.float32

NC = 2
NS = 16
LANES = 16
CH = 128


def _pad8(d: int) -> int:
    return ((d + 7) // 8) * 8


def _mesh():
    return plsc.VectorSubcoreMesh(core_axis_name="c", subcore_axis_name="s")


_SPMEM_WORDS = 2097151


def _pick_group(t, acc_fp, row_fp, npad, n_idx, n_rows, fixed_words):
    budget = (_SPMEM_WORDS - npad * acc_fp) // NS - 64
    epw = t * CH
    for kch in (2560, 1280, 640, 512, 320, 256, 128):
        if epw % kch:
            continue
        g = epw // kch
        if g < 2 or g % 2:
            continue
        need = n_rows * kch * row_fp + n_idx * epw + fixed_words
        if need <= budget:
            return kch, g
    raise ValueError("no feasible group size")


def _make_sc_degree(n, t):
    npad = n + 8
    rpt = n // NS
    kch, g = _pick_group(t, LANES, LANES, npad, 1, 1, rpt * LANES)

    @functools.partial(
        pl.kernel,
        out_type=jax.ShapeDtypeStruct((NC, n, LANES), F32),
        mesh=_mesh(),
        compiler_params=pltpu.CompilerParams(use_tc_tiling_on_sc=False),
        scratch_types=[
            pltpu.VMEM((g, kch), jnp.int32),
            pltpu.VMEM((kch, LANES), F32),
            pltpu.VMEM_SHARED((npad, LANES), F32),
            pltpu.SemaphoreType.DMA,
        ],
    )
    def deg_kernel(dst_hbm, zeros_hbm, ones_hbm, out_hbm,
                   dst_v, ones_v, acc_sh, sem):
        c = lax.axis_index("c")
        s = lax.axis_index("s")
        wid = c * NS + s
        pltpu.sync_copy(dst_hbm.at[wid], dst_v)
        pltpu.sync_copy(ones_hbm, ones_v)
        sl = pl.ds(s * rpt, rpt)
        pltpu.sync_copy(zeros_hbm.at[sl], acc_sh.at[sl])
        plsc.subcore_barrier()

        def body(gg, carry):
            cp = pltpu.make_async_copy(ones_v, acc_sh.at[dst_v.at[gg]], sem)
            cp.start(add=True)
            cp.wait()
            return carry

        lax.fori_loop(0, g, body, 0)
        plsc.subcore_barrier()
        pltpu.sync_copy(acc_sh.at[sl], out_hbm.at[c, sl])

    return deg_kernel


def _agg_config(t, fp, npad):
    budget = (_SPMEM_WORDS - 2 * npad * fp) // NS - 64
    epw = t * CH
    for kch in (1280, 640, 512, 320, 256, 128):
        if epw % kch:
            continue
        g = epw // kch
        if g < 2 or g % 2:
            continue
        need = 2 * kch * fp + 4 * kch
        if need <= budget:
            return kch, g
    raise ValueError("no feasible group size")


def _make_sc_agg(n, t, fp):
    npad = n + 8
    rpt = n // NS
    kch, g = _agg_config(t, fp, npad)

    @functools.partial(
        pl.kernel,
        out_type=jax.ShapeDtypeStruct((NC, n, fp), F32),
        mesh=_mesh(),
        compiler_params=pltpu.CompilerParams(use_tc_tiling_on_sc=False),
        scratch_types=[
            pltpu.VMEM((2, kch), jnp.int32),
            pltpu.VMEM((2, kch), jnp.int32),
            pltpu.VMEM((kch, fp), F32),
            pltpu.VMEM((kch, fp), F32),
            pltpu.VMEM_SHARED((npad, fp), F32),
            pltpu.VMEM_SHARED((npad, fp), F32),
            pltpu.SemaphoreType.DMA,
            pltpu.SemaphoreType.DMA,
            pltpu.SemaphoreType.DMA,
            pltpu.SemaphoreType.DMA,
            pltpu.SemaphoreType.DMA,
            pltpu.SemaphoreType.DMA,
        ],
    )
    def agg_kernel(idx_hbm, hs_hbm, zeros_hbm, out_hbm,
                   idx0, idx1, rows0, rows1, table_sh, acc_sh,
                   is0, is1, gs0, gs1, ss0, ss1):
        c = lax.axis_index("c")
        s = lax.axis_index("s")
        wid = c * NS + s

        sl = pl.ds(s * rpt, rpt)
        pltpu.sync_copy(hs_hbm.at[sl], table_sh.at[sl])

        @pl.when(c == 0)
        def _():
            pltpu.sync_copy(hs_hbm.at[sl], acc_sh.at[sl])

        @pl.when(c != 0)
        def _():
            pltpu.sync_copy(zeros_hbm.at[sl], acc_sh.at[sl])

        plsc.subcore_barrier()

        idx = (idx0, idx1)
        rows = (rows0, rows1)
        isem = (is0, is1)
        gsem = (gs0, gs1)
        ssem = (ss0, ss1)

        def idxc(gg, j):
            return pltpu.make_async_copy(idx_hbm.at[wid, gg], idx[j], isem[j])

        def gather(gg, j):
            return pltpu.make_async_copy(
                table_sh.at[idx[j].at[0]], rows[j], gsem[j])

        def scat(gg, j):
            return pltpu.make_async_copy(
                rows[j], acc_sh.at[idx[j].at[1]], ssem[j])

        idxc(0, 0).start()
        idxc(0, 0).wait()
        gather(0, 0).start()
        gather(0, 0).wait()
        scat(0, 0).start(add=True)
        idxc(1, 1).start()
        idxc(1, 1).wait()
        gather(1, 1).start()
        gather(1, 1).wait()
        scat(0, 0).wait()
        scat(1, 1).start(add=True)

        def body(i, carry):
            g0 = 2 * i
            idxc(g0, 0).start()
            idxc(g0, 0).wait()
            gather(g0, 0).start()
            gather(g0, 0).wait()
            scat(g0 - 1, 1).wait()
            scat(g0, 0).start(add=True)
            idxc(g0 + 1, 1).start()
            idxc(g0 + 1, 1).wait()
            gather(g0 + 1, 1).start()
            gather(g0 + 1, 1).wait()
            scat(g0, 0).wait()
            scat(g0 + 1, 1).start(add=True)
            return carry

        lax.fori_loop(1, g // 2, body, 0)
        scat(g - 1, 1).wait()
        plsc.subcore_barrier()
        pltpu.sync_copy(acc_sh.at[sl], out_hbm.at[c, sl])

    return agg_kernel


_RB = 2000


def _tc_mm0(x, w0):
    n, fin = x.shape
    fout = w0.shape[1]

    def body(x_r, w_r, h_r):
        h_r[...] = jnp.dot(x_r[...], w_r[...], preferred_element_type=F32)

    return pl.pallas_call(
        body,
        grid=(n // _RB,),
        in_specs=[
            pl.BlockSpec((_RB, fin), lambda i: (i, 0)),
            pl.BlockSpec((fin, fout), lambda i: (0, 0)),
        ],
        out_specs=pl.BlockSpec((_RB, fout), lambda i: (i, 0)),
        out_shape=jax.ShapeDtypeStruct((n, fout), F32),
    )(x, w0)


def _tc_scale(d0, d1, h):
    n, fout = h.shape

    def body(d0_r, d1_r, h_r, dinv_r, hs_r):
        dinv = 1.0 / jnp.sqrt(1.0 + d0_r[...] + d1_r[...])
        dinv_r[...] = dinv
        hs_r[...] = h_r[...] * dinv

    return pl.pallas_call(
        body,
        grid=(n // _RB,),
        in_specs=[
            pl.BlockSpec((_RB, 1), lambda i: (i, 0)),
            pl.BlockSpec((_RB, 1), lambda i: (i, 0)),
            pl.BlockSpec((_RB, fout), lambda i: (i, 0)),
        ],
        out_specs=[
            pl.BlockSpec((_RB, 1), lambda i: (i, 0)),
            pl.BlockSpec((_RB, fout), lambda i: (i, 0)),
        ],
        out_shape=[
            jax.ShapeDtypeStruct((n, 1), F32),
            jax.ShapeDtypeStruct((n, fout), F32),
        ],
    )(d0, d1, h)


def _tc_mid(agg, dinv, bvec, scale, beta, w):
    _, n, fin = agg.shape
    fout = w.shape[1]

    def body(agg_r, dinv_r, b_r, s_r, bt_r, w_r, hs_r):
        dinv = dinv_r[...]
        y = (agg_r[0] + agg_r[1]) * dinv + b_r[...]
        act = jnp.where(y > 0, y, jnp.exp(y) - 1.0)
        z = act * s_r[...] + bt_r[...]
        h = jnp.dot(z, w_r[...], preferred_element_type=F32)
        hs_r[...] = h * dinv

    return pl.pallas_call(
        body,
        grid=(n // _RB,),
        in_specs=[
            pl.BlockSpec((2, _RB, fin), lambda i: (0, i, 0)),
            pl.BlockSpec((_RB, 1), lambda i: (i, 0)),
            pl.BlockSpec((1, fin), lambda i: (0, 0)),
            pl.BlockSpec((1, fin), lambda i: (0, 0)),
            pl.BlockSpec((1, fin), lambda i: (0, 0)),
            pl.BlockSpec((fin, fout), lambda i: (0, 0)),
        ],
        out_specs=pl.BlockSpec((_RB, fout), lambda i: (i, 0)),
        out_shape=jax.ShapeDtypeStruct((n, fout), F32),
    )(agg, dinv, bvec, scale, beta, w)


def _tc_final(agg, dinv, bvec):
    _, n, fin = agg.shape

    def body(agg_r, dinv_r, b_r, y_r):
        y_r[...] = (agg_r[0] + agg_r[1]) * dinv_r[...] + b_r[...]

    return pl.pallas_call(
        body,
        grid=(n // _RB,),
        in_specs=[
            pl.BlockSpec((2, _RB, fin), lambda i: (0, i, 0)),
            pl.BlockSpec((_RB, 1), lambda i: (i, 0)),
            pl.BlockSpec((1, fin), lambda i: (0, 0)),
        ],
        out_specs=pl.BlockSpec((_RB, fin), lambda i: (i, 0)),
        out_shape=jax.ShapeDtypeStruct((n, fin), F32),
    )(agg, dinv, bvec)


def kernel(x, edge_index, Ws, bs, gammas, betas):
    n = x.shape[0]
    e = edge_index.shape[1]
    nl = len(Ws)
    dims = [x.shape[1]] + [w.shape[1] for w in Ws]
    fps = [_pad8(d) for d in dims]

    t = math.ceil(e / (NC * NS * CH))
    t = ((t + 7) // 8) * 8
    e_pad = NC * NS * t * CH

    src = jnp.concatenate(
        [edge_index[0], jnp.zeros((e_pad - e,), edge_index.dtype)]
    ).reshape(NC * NS, t * CH)
    dst = jnp.concatenate(
        [edge_index[1], jnp.full((e_pad - e,), n, edge_index.dtype)]
    ).reshape(NC * NS, t * CH)

    npad = n + 8

    def grouped(a, kch):
        return a.reshape(NC * NS, (t * CH) // kch, kch)

    def interleaved(kch):
        gg = (t * CH) // kch
        return jnp.stack(
            [src.reshape(NC * NS, gg, kch), dst.reshape(NC * NS, gg, kch)],
            axis=2)

    kch_deg, _ = _pick_group(t, LANES, LANES, npad, 1, 1, (n // NS) * LANES)
    degp = _make_sc_degree(n, t)(
        grouped(dst, kch_deg), jnp.zeros((n, LANES), F32),
        jnp.ones((kch_deg, LANES), F32))
    d0 = degp[0, :, 0:1]
    d1 = degp[1, :, 0:1]

    xp = jnp.pad(x, ((0, 0), (0, fps[0] - dims[0])))
    Wp = [jnp.pad(Ws[i], ((0, fps[i] - dims[i]), (0, fps[i + 1] - dims[i + 1])))
          for i in range(nl)]
    bp = [jnp.pad(bs[i], (0, fps[i + 1] - dims[i + 1])).reshape(1, -1)
          for i in range(nl)]
    inv_bn = 1.0 / np.sqrt(1.0 + 1e-5)
    scalep = [(jnp.pad(gammas[i], (0, fps[i + 1] - dims[i + 1])) * inv_bn
               ).reshape(1, -1) for i in range(nl - 1)]
    betap = [jnp.pad(betas[i], (0, fps[i + 1] - dims[i + 1])).reshape(1, -1)
             for i in range(nl - 1)]

    h0 = _tc_mm0(xp, Wp[0])
    dinv, hs = _tc_scale(d0, d1, h0)

    for i in range(nl - 1):
        fp = fps[i + 1]
        kch, _ = _agg_config(t, fp, npad)
        agg = _make_sc_agg(n, t, fp)(
            interleaved(kch), hs, jnp.zeros((n, fp), F32))
        hs = _tc_mid(agg, dinv, bp[i], scalep[i], betap[i], Wp[i + 1])

    fp = fps[nl]
    kch, _ = _agg_config(t, fp, npad)
    agg = _make_sc_agg(n, t, fp)(
        interleaved(kch), hs, jnp.zeros((n, fp), F32))
    y = _tc_final(agg, dinv, bp[nl - 1])
    return y[:, :dims[nl]]
